# Initial kernel scaffold; baseline (speedup 1.0000x reference)
import jax
import jax.numpy as jnp
from jax import lax
from jax.experimental import pallas as pl
from jax.experimental.pallas import tpu as pltpu
from jax.experimental.pallas import tpu_sc as plsc

N = 50000
NPAD = 50048
F = 1433
H = 128
E = 800000
NCC = 2
NSS = 16
NW = NCC * NSS
ECH = 2500
CH_R, CH_C = 20, 125
CPT = E // (NW * ECH)
NCHUNKS = E // ECH
STRIPE = NPAD // NSS
ZR = STRIPE // 8
BM = 2000
GRID = N // BM


def _sc_degree(dst3, ones_h, zeros_h):
    mesh = plsc.VectorSubcoreMesh(core_axis_name="c", subcore_axis_name="s")

    def body(dst_hbm, ones_hbm, zeros_hbm, out0, out1, idx_v, ones_v, stage_v, slab):
        c = lax.axis_index("c")
        s = lax.axis_index("s")
        wid = s * NCC + c
        pltpu.sync_copy(ones_hbm, ones_v)
        pltpu.sync_copy(zeros_hbm, stage_v)
        pltpu.sync_copy(stage_v, slab.at[pl.ds(s * STRIPE, STRIPE)])
        plsc.subcore_barrier()

        def chunk(j, carry):
            pltpu.sync_copy(dst_hbm.at[wid * CPT + j], idx_v)
            pltpu.sync_copy(ones_v, slab.at[idx_v], add=True)
            return carry

        lax.fori_loop(0, CPT, chunk, 0)
        plsc.subcore_barrier()
        pltpu.sync_copy(slab.at[pl.ds(s * STRIPE, STRIPE)], stage_v)

        @pl.when(c == 0)
        def _():
            pltpu.sync_copy(stage_v, out0.at[pl.ds(s * STRIPE, STRIPE)])

        @pl.when(c == 1)
        def _():
            pltpu.sync_copy(stage_v, out1.at[pl.ds(s * STRIPE, STRIPE)])

    f = pl.kernel(
        body,
        out_type=[jax.ShapeDtypeStruct((NPAD,), jnp.float32)] * 2,
        mesh=mesh,
        scratch_types=[
            pltpu.VMEM((CH_R, CH_C), jnp.int32),
            pltpu.VMEM((CH_R, CH_C), jnp.float32),
            pltpu.VMEM((STRIPE,), jnp.float32),
            pltpu.VMEM_SHARED((NPAD,), jnp.float32),
        ],
    )
    return f(dst3, ones_h, zeros_h)


def _sc_prop(src3, dst3, tables, D, zeros_h):
    NT = len(tables)
    mesh = plsc.VectorSubcoreMesh(core_axis_name="c", subcore_axis_name="s")

    def body(src_hbm, dst_hbm, *rest):
        tabs = rest[:NT]
        zeros_hbm = rest[NT]
        outs = rest[NT + 1: NT + 1 + 2 * NT]
        sidx, didx, rows_v, zero_v, stage_v, slab, sem = rest[NT + 1 + 2 * NT:]
        c = lax.axis_index("c")
        s = lax.axis_index("s")
        wid = s * NCC + c
        pltpu.sync_copy(zeros_hbm, zero_v)
        for t in range(NT):
            for z in range(8):
                pltpu.sync_copy(zero_v, slab.at[pl.ds(s * STRIPE + z * ZR, ZR)])
            plsc.subcore_barrier()

            def chunk(j, carry, t=t):
                pltpu.sync_copy(src_hbm.at[wid * CPT + j], sidx)
                pltpu.sync_copy(dst_hbm.at[wid * CPT + j], didx)
                pltpu.async_copy(tabs[t].at[sidx], rows_v, sem).wait()
                pltpu.sync_copy(rows_v, slab.at[didx], add=True)
                return carry

            lax.fori_loop(0, CPT, chunk, 0)
            plsc.subcore_barrier()
            for z in range(8):
                sl = pl.ds(s * STRIPE + z * ZR, ZR)
                pltpu.sync_copy(slab.at[sl], stage_v)

                @pl.when(c == 0)
                def _(t=t, sl=sl):
                    pltpu.sync_copy(stage_v, outs[t].at[sl])

                @pl.when(c == 1)
                def _(t=t, sl=sl):
                    pltpu.sync_copy(stage_v, outs[NT + t].at[sl])

            plsc.subcore_barrier()

    f = pl.kernel(
        body,
        out_type=[jax.ShapeDtypeStruct((NPAD, D), jnp.float32)] * (2 * NT),
        mesh=mesh,
        scratch_types=[
            pltpu.VMEM((CH_R, CH_C), jnp.int32),
            pltpu.VMEM((CH_R, CH_C), jnp.int32),
            pltpu.VMEM((CH_R, CH_C, D), jnp.float32),
            pltpu.VMEM((ZR, D), jnp.float32),
            pltpu.VMEM((ZR, D), jnp.float32),
            pltpu.VMEM_SHARED((NPAD, D), jnp.float32),
            pltpu.SemaphoreType.DMA,
        ],
    )
    return f(src3, dst3, *tables, zeros_h)


def _tc_a(x, W1, d0, d1):
    def body(x_ref, w_ref, d0_ref, d1_ref, o0, o1, o2, o3):
        dinv = lax.rsqrt(d0_ref[...] + d1_ref[...] + 1.0)
        xw = lax.dot_general(
            x_ref[...], w_ref[...], (((1,), (0,)), ((), ())),
            precision=lax.Precision.HIGHEST,
            preferred_element_type=jnp.float32,
        )
        vs = xw * dinv
        o0[...] = vs[:, 0:32]
        o1[...] = vs[:, 32:64]
        o2[...] = vs[:, 64:96]
        o3[...] = vs[:, 96:128]

    return pl.pallas_call(
        body,
        grid=(GRID,),
        in_specs=[
            pl.BlockSpec((BM, F), lambda m: (m, 0)),
            pl.BlockSpec((F, H), lambda m: (0, 0)),
            pl.BlockSpec((BM, 1), lambda m: (m, 0)),
            pl.BlockSpec((BM, 1), lambda m: (m, 0)),
        ],
        out_specs=[pl.BlockSpec((BM, 32), lambda m: (m, 0))] * 4,
        out_shape=[jax.ShapeDtypeStruct((N, 32), jnp.float32)] * 4,
    )(x, W1, d0, d1)


def _tc_b(ps, vs1s, d0, d1, b1r, W2p):
    def body(p00, p01, p02, p03, p10, p11, p12, p13,
             v0, v1, v2, v3, d0_ref, d1_ref, b_ref, w_ref, o_ref):
        dinv = lax.rsqrt(d0_ref[...] + d1_ref[...] + 1.0)
        agg = jnp.concatenate(
            [p00[...] + p10[...] + v0[...],
             p01[...] + p11[...] + v1[...],
             p02[...] + p12[...] + v2[...],
             p03[...] + p13[...] + v3[...]], axis=1)
        h = jnp.maximum(agg * dinv + b_ref[...], 0.0)
        hw = lax.dot_general(
            h, w_ref[...], (((1,), (0,)), ((), ())),
            precision=lax.Precision.HIGHEST,
            preferred_element_type=jnp.float32,
        )
        o_ref[...] = hw * dinv

    row32 = pl.BlockSpec((BM, 32), lambda m: (m, 0))
    return pl.pallas_call(
        body,
        grid=(GRID,),
        in_specs=[row32] * 12 + [
            pl.BlockSpec((BM, 1), lambda m: (m, 0)),
            pl.BlockSpec((BM, 1), lambda m: (m, 0)),
            pl.BlockSpec((1, H), lambda m: (0, 0)),
            pl.BlockSpec((H, 16), lambda m: (0, 0)),
        ],
        out_specs=pl.BlockSpec((BM, 16), lambda m: (m, 0)),
        out_shape=jax.ShapeDtypeStruct((N, 16), jnp.float32),
    )(*ps, *vs1s, d0, d1, b1r, W2p)


def _tc_c(q0, q1, vs2, d0, d1, b2p):
    def body(q0_ref, q1_ref, v_ref, d0_ref, d1_ref, b_ref, o_ref):
        dinv = lax.rsqrt(d0_ref[...] + d1_ref[...] + 1.0)
        o_ref[...] = (q0_ref[...] + q1_ref[...] + v_ref[...]) * dinv + b_ref[...]

    row16 = pl.BlockSpec((BM, 16), lambda m: (m, 0))
    return pl.pallas_call(
        body,
        grid=(GRID,),
        in_specs=[row16, row16, row16,
                  pl.BlockSpec((BM, 1), lambda m: (m, 0)),
                  pl.BlockSpec((BM, 1), lambda m: (m, 0)),
                  pl.BlockSpec((1, 16), lambda m: (0, 0))],
        out_specs=row16,
        out_shape=jax.ShapeDtypeStruct((N, 16), jnp.float32),
    )(q0, q1, vs2, d0, d1, b2p)


def kernel(x, edge_index, W1, b1, W2, b2):
    src = edge_index[0].astype(jnp.int32).reshape(NCHUNKS, CH_R, CH_C)
    dst = edge_index[1].astype(jnp.int32).reshape(NCHUNKS, CH_R, CH_C)
    ones_h = jnp.ones((CH_R, CH_C), jnp.float32)
    zeros_deg = jnp.zeros((STRIPE,), jnp.float32)
    zeros32 = jnp.zeros((ZR, 32), jnp.float32)
    zeros16 = jnp.zeros((ZR, 16), jnp.float32)

    d0, d1 = _sc_degree(dst, ones_h, zeros_deg)
    d0 = d0.reshape(NPAD, 1)
    d1 = d1.reshape(NPAD, 1)

    vs1 = _tc_a(x, W1, d0, d1)
    ps = _sc_prop(src, dst, list(vs1), 32, zeros32)

    b1r = b1.reshape(1, H)
    W2p = jnp.pad(W2, ((0, 0), (0, 9)))
    vs2 = _tc_b(list(ps), list(vs1), d0, d1, b1r, W2p)

    qs = _sc_prop(src, dst, [vs2], 16, zeros16)
    b2p = jnp.pad(b2, (0, 9)).reshape(1, 16)
    out16 = _tc_c(qs[0], qs[1], vs2, d0, d1, b2p)
    return out16[:, :7]


# speedup vs baseline: 4.5880x; 4.5880x over previous
import jax
import jax.numpy as jnp
from jax import lax
from jax.experimental import pallas as pl
from jax.experimental.pallas import tpu as pltpu
from jax.experimental.pallas import tpu_sc as plsc

N = 50000
NPAD = 51200
F = 1433
H = 128
E = 800000
E_PAD = 819200
NCC = 2
NSS = 16
NW = NCC * NSS
ECH_D = 640
CPT_D = E_PAD // (NW * ECH_D)
NCH_D = E_PAD // ECH_D
ECH = 128
CPT = E_PAD // (NW * ECH)
NCH_P = E_PAD // ECH
DSTRIPE = NPAD // NSS
RANGE = NPAD // 4
DUMMY = 1024
SLAB = RANGE + DUMMY
ZSTRIPE = RANGE // NSS
ZW = 80
BM = 2000
GRID = N // BM


def _sc_degree(dst3, ones_h, zeros_h):
    mesh = plsc.VectorSubcoreMesh(core_axis_name="c", subcore_axis_name="s")

    def body(dst_hbm, ones_hbm, zeros_hbm, out0, out1, idx_v, ones_v, stage_v, slab):
        c = lax.axis_index("c")
        s = lax.axis_index("s")
        wid = s * NCC + c
        pltpu.sync_copy(ones_hbm, ones_v)
        pltpu.sync_copy(zeros_hbm, stage_v)
        pltpu.sync_copy(stage_v, slab.at[pl.ds(s * DSTRIPE, DSTRIPE)])
        plsc.subcore_barrier()

        def chunk(j, carry):
            pltpu.sync_copy(dst_hbm.at[wid * CPT_D + j], idx_v)
            pltpu.sync_copy(ones_v, slab.at[idx_v], add=True)
            return carry

        lax.fori_loop(0, CPT_D, chunk, 0)
        plsc.subcore_barrier()
        pltpu.sync_copy(slab.at[pl.ds(s * DSTRIPE, DSTRIPE)], stage_v)

        @pl.when(c == 0)
        def _():
            pltpu.sync_copy(stage_v, out0.at[pl.ds(s * DSTRIPE, DSTRIPE)])

        @pl.when(c == 1)
        def _():
            pltpu.sync_copy(stage_v, out1.at[pl.ds(s * DSTRIPE, DSTRIPE)])

    f = pl.kernel(
        body,
        out_type=[jax.ShapeDtypeStruct((NPAD,), jnp.float32)] * 2,
        mesh=mesh,
        scratch_types=[
            pltpu.VMEM((ECH_D,), jnp.int32),
            pltpu.VMEM((ECH_D,), jnp.float32),
            pltpu.VMEM((DSTRIPE,), jnp.float32),
            pltpu.VMEM_SHARED((NPAD,), jnp.float32),
        ],
    )
    return f(dst3, ones_h, zeros_h)


def _sc_prop(src3, dst3, tab, zeros_h):
    mesh = plsc.VectorSubcoreMesh(core_axis_name="c", subcore_axis_name="s")

    def body(src_hbm, dst_hbm, tab_hbm, zeros_hbm, out0, out1,
             sidx, didx, didx2, rows_v, slab, sem):
        c = lax.axis_index("c")
        s = lax.axis_index("s")
        wid = s * NCC + c
        for t in range(4):
            lo = t * RANGE
            pltpu.sync_copy(zeros_hbm, rows_v.at[pl.ds(0, ZW)])
            for z in range(ZSTRIPE // ZW):
                pltpu.sync_copy(rows_v.at[pl.ds(0, ZW)],
                                slab.at[pl.ds(s * ZSTRIPE + z * ZW, ZW)])
            plsc.subcore_barrier()

            def chunk(j, carry, lo=lo):
                pltpu.sync_copy(src_hbm.at[wid * CPT + j], sidx)
                pltpu.sync_copy(dst_hbm.at[wid * CPT + j], didx)

                def remap(i, cc, lo=lo, j=j):
                    d = didx[pl.ds(i * 16, 16)]
                    inr = (d >= lo) & (d < lo + RANGE)
                    spread = (j * ECH + i * 16 + lax.iota(jnp.int32, 16)) & (DUMMY - 1)
                    didx2[pl.ds(i * 16, 16)] = jnp.where(inr, d - lo, RANGE + spread)
                    return cc

                lax.fori_loop(0, ECH // 16, remap, 0)
                pltpu.async_copy(tab_hbm.at[sidx], rows_v, sem).wait()
                pltpu.sync_copy(rows_v, slab.at[didx2], add=True)
                return carry

            lax.fori_loop(0, CPT, chunk, 0)
            plsc.subcore_barrier()
            for k in range(ZSTRIPE // ZW):
                pltpu.sync_copy(slab.at[pl.ds(s * ZSTRIPE + k * ZW, ZW)],
                                rows_v.at[pl.ds(0, ZW)])

                @pl.when(c == 0)
                def _(t=t, k=k):
                    pltpu.sync_copy(rows_v.at[pl.ds(0, ZW)],
                                    out0.at[pl.ds(t * RANGE + s * ZSTRIPE + k * ZW, ZW)])

                @pl.when(c == 1)
                def _(t=t, k=k):
                    pltpu.sync_copy(rows_v.at[pl.ds(0, ZW)],
                                    out1.at[pl.ds(t * RANGE + s * ZSTRIPE + k * ZW, ZW)])

            plsc.subcore_barrier()

    f = pl.kernel(
        body,
        out_type=[jax.ShapeDtypeStruct((NPAD, H), jnp.float32)] * 2,
        mesh=mesh,
        scratch_types=[
            pltpu.VMEM((ECH,), jnp.int32),
            pltpu.VMEM((ECH,), jnp.int32),
            pltpu.VMEM((ECH,), jnp.int32),
            pltpu.VMEM((ECH, H), jnp.float32),
            pltpu.VMEM_SHARED((SLAB, H), jnp.float32),
            pltpu.SemaphoreType.DMA,
        ],
    )
    return f(src3, dst3, tab, zeros_h)


def _tc_a(x, W1, d0, d1):
    def body(x_ref, w_ref, d0_ref, d1_ref, o_ref):
        dinv = lax.rsqrt(d0_ref[...] + d1_ref[...] + 1.0)
        xw = lax.dot_general(
            x_ref[...], w_ref[...], (((1,), (0,)), ((), ())),
            precision=lax.Precision.HIGHEST,
            preferred_element_type=jnp.float32,
        )
        o_ref[...] = xw * dinv

    return pl.pallas_call(
        body,
        grid=(GRID,),
        in_specs=[
            pl.BlockSpec((BM, F), lambda m: (m, 0)),
            pl.BlockSpec((F, H), lambda m: (0, 0)),
            pl.BlockSpec((BM, 1), lambda m: (m, 0)),
            pl.BlockSpec((BM, 1), lambda m: (m, 0)),
        ],
        out_specs=pl.BlockSpec((BM, H), lambda m: (m, 0)),
        out_shape=jax.ShapeDtypeStruct((N, H), jnp.float32),
    )(x, W1, d0, d1)


def _tc_b(p0, p1, vs1, d0, d1, b1r):
    def body(p0_ref, p1_ref, v_ref, d0_ref, d1_ref, b_ref, o_ref):
        dinv = lax.rsqrt(d0_ref[...] + d1_ref[...] + 1.0)
        agg = p0_ref[...] + p1_ref[...] + v_ref[...]
        h = jnp.maximum(agg * dinv + b_ref[...], 0.0)
        o_ref[...] = h * dinv

    rowH = pl.BlockSpec((BM, H), lambda m: (m, 0))
    return pl.pallas_call(
        body,
        grid=(GRID,),
        in_specs=[rowH, rowH, rowH,
                  pl.BlockSpec((BM, 1), lambda m: (m, 0)),
                  pl.BlockSpec((BM, 1), lambda m: (m, 0)),
                  pl.BlockSpec((1, H), lambda m: (0, 0))],
        out_specs=rowH,
        out_shape=jax.ShapeDtypeStruct((N, H), jnp.float32),
    )(p0, p1, vs1, d0, d1, b1r)


def _tc_c(q0, q1, vs2, d0, d1, W2p, b2p):
    def body(q0_ref, q1_ref, v_ref, d0_ref, d1_ref, w_ref, b_ref, o_ref):
        dinv = lax.rsqrt(d0_ref[...] + d1_ref[...] + 1.0)
        agg = (q0_ref[...] + q1_ref[...] + v_ref[...]) * dinv
        o_ref[...] = lax.dot_general(
            agg, w_ref[...], (((1,), (0,)), ((), ())),
            precision=lax.Precision.HIGHEST,
            preferred_element_type=jnp.float32,
        ) + b_ref[...]

    rowH = pl.BlockSpec((BM, H), lambda m: (m, 0))
    return pl.pallas_call(
        body,
        grid=(GRID,),
        in_specs=[rowH, rowH, rowH,
                  pl.BlockSpec((BM, 1), lambda m: (m, 0)),
                  pl.BlockSpec((BM, 1), lambda m: (m, 0)),
                  pl.BlockSpec((H, 16), lambda m: (0, 0)),
                  pl.BlockSpec((1, 16), lambda m: (0, 0))],
        out_specs=pl.BlockSpec((BM, 16), lambda m: (m, 0)),
        out_shape=jax.ShapeDtypeStruct((N, 16), jnp.float32),
    )(q0, q1, vs2, d0, d1, W2p, b2p)


def kernel(x, edge_index, W1, b1, W2, b2):
    src = edge_index[0].astype(jnp.int32)
    dst = edge_index[1].astype(jnp.int32)
    pad_n = E_PAD - E
    ar = jnp.arange(pad_n, dtype=jnp.int32)
    src_p = jnp.concatenate([src, (ar * 2503) % N])
    dst_p = jnp.concatenate([dst, N + (ar % (NPAD - N))])
    dst_d = dst_p.reshape(NCH_D, ECH_D)
    src3 = src_p.reshape(NCH_P, ECH)
    dst3 = dst_p.reshape(NCH_P, ECH)

    ones_h = jnp.ones((ECH_D,), jnp.float32)
    zeros_deg = jnp.zeros((DSTRIPE,), jnp.float32)
    zeros_h = jnp.zeros((ZW, H), jnp.float32)

    d0, d1 = _sc_degree(dst_d, ones_h, zeros_deg)
    d0 = d0.reshape(NPAD, 1)
    d1 = d1.reshape(NPAD, 1)

    vs1 = _tc_a(x, W1, d0, d1)
    p0, p1 = _sc_prop(src3, dst3, vs1, zeros_h)

    b1r = b1.reshape(1, H)
    vs2 = _tc_b(p0, p1, vs1, d0, d1, b1r)

    q0, q1 = _sc_prop(src3, dst3, vs2, zeros_h)
    W2p = jnp.pad(W2, ((0, 0), (0, 9)))
    b2p = jnp.pad(b2, (0, 9)).reshape(1, 16)
    out16 = _tc_c(q0, q1, vs2, d0, d1, W2p, b2p)
    return out16[:, :7]
